# baseline (speedup 1.0000x reference)
import functools

import jax
import jax.numpy as jnp
from jax import lax
from jax.experimental import pallas as pl
from jax.experimental.pallas import tpu as pltpu
from jax.experimental.pallas import tpu_sc as plsc

NCB, NPOINT, NDIM = 8, 4096, 64
CB = 1024
P = 512
NPB = NPOINT // P

_NC, _NS = 2, 16
_NW = _NC * _NS
_BPW = NCB * NPOINT // _NW
_ICH = 128
_NCH = _BPW // _ICH


def _vq_body(x_ref, cbt_ref, dist_ref, oh_ref, idx_ref):
    g = pl.program_id(0)
    x = x_ref[0]
    cbt = cbt_ref[0]
    xn = jnp.sum(x * x, axis=1, keepdims=True)
    cn = jnp.sum(cbt * cbt, axis=0, keepdims=True)
    prod = lax.dot_general(x, cbt, (((1,), (0,)), ((), ())),
                           preferred_element_type=jnp.float32)
    dist = (xn + cn - 2.0 * prod) * (1.0 / NDIM)

    iota = lax.broadcasted_iota(jnp.int32, (P, CB), 1).astype(jnp.float32)
    m = jnp.min(dist, axis=1, keepdims=True)
    cand = jnp.where(dist == m, iota, float(CB))
    idx = jnp.min(cand, axis=1, keepdims=True)
    one_hot = (iota == idx).astype(jnp.float32)

    dist_ref[0] = dist
    oh_ref[0] = one_hot
    idx_ref[0] = idx.astype(jnp.int32) + g * CB


def _vq_tc(x, cb_t):
    return pl.pallas_call(
        _vq_body,
        grid=(NCB, NPB),
        in_specs=[
            pl.BlockSpec((1, P, NDIM), lambda g, p: (g, p, 0)),
            pl.BlockSpec((1, NDIM, CB), lambda g, p: (g, 0, 0)),
        ],
        out_specs=[
            pl.BlockSpec((1, P, CB), lambda g, p: (g, p, 0)),
            pl.BlockSpec((1, P, CB), lambda g, p: (g, p, 0)),
            pl.BlockSpec((1, P, 1), lambda g, p: (g, p, 0)),
        ],
        out_shape=[
            jax.ShapeDtypeStruct((NCB, NPOINT, CB), jnp.float32),
            jax.ShapeDtypeStruct((NCB, NPOINT, CB), jnp.float32),
            jax.ShapeDtypeStruct((NCB, NPOINT, 1), jnp.int32),
        ],
        compiler_params=pltpu.CompilerParams(
            dimension_semantics=("parallel", "arbitrary")),
    )(x, cb_t)


_sc_mesh = plsc.VectorSubcoreMesh(core_axis_name="c", subcore_axis_name="s")


@functools.partial(
    pl.kernel,
    mesh=_sc_mesh,
    out_type=jax.ShapeDtypeStruct((NCB * NPOINT, NDIM), jnp.float32),
    scratch_types=[
        pltpu.VMEM((_NCH, _ICH), jnp.int32),
        pltpu.VMEM((_BPW, NDIM), jnp.float32),
        pltpu.SemaphoreType.DMA,
    ],
    compiler_params=pltpu.CompilerParams(use_tc_tiling_on_sc=False),
)
def _sc_gather(table_hbm, idx_hbm, out_hbm, idx_v, rows_v, sem):
    wid = lax.axis_index("s") * _NC + lax.axis_index("c")
    pltpu.sync_copy(idx_hbm.at[wid], idx_v)
    copies = [
        pltpu.async_copy(table_hbm.at[idx_v.at[j]],
                         rows_v.at[pl.ds(j * _ICH, _ICH)], sem)
        for j in range(_NCH)
    ]
    for c in copies:
        c.wait()
    pltpu.sync_copy(rows_v, out_hbm.at[pl.ds(wid * _BPW, _BPW)])


def kernel(x, code_book):
    cb_t = jnp.transpose(code_book, (0, 2, 1))
    dist, one_hot, idx = _vq_tc(x, cb_t)
    table = code_book.reshape(NCB * CB, NDIM)
    idx3 = idx.reshape(_NW, _NCH, _ICH)
    x_hat = _sc_gather(table, idx3).reshape(NCB, NPOINT, NDIM)
    return (x_hat, one_hot, dist)
